# baseline (speedup 1.0000x reference)
import jax
import jax.numpy as jnp
from jax import lax
from jax.experimental import pallas as pl
from jax.experimental.pallas import tpu as pltpu
from jax.experimental.pallas import tpu_sc as plsc

N = 10000
E = 160000
IN_DIM = 256
HIDDEN = 512
OUT_DIM = 256
SPE = 64
M = 8
ALPHA = 0.5

NP_ = 10240
RB = 512
GRID = NP_ // RB
NSC = 2
NTILE = 16
TR = NP_ // NTILE
K = 128
NIT = 80
NH = 40
EP = NTILE * NIT * K


def _relu(x):
    return jnp.maximum(x, 0.0)


def _rsqrt_deg(cnt_blk):
    return lax.rsqrt(jnp.maximum(cnt_blk[:, :1], 1.0))


def _dot(a, b):
    return jnp.dot(a, b, preferred_element_type=jnp.float32)


def _embed_body(ev8, evec, se, h, cs, wh, bh,
                pw1, pb1, pw2, pb2, pw3, pb3, pw4, pb4,
                sw1, sb1, sw2, sb2, sw3, sb3, sw4, sb4,
                w1g, out):
    dout = _rsqrt_deg(cs[...])
    x = _relu(_dot(evec[...], pw1[M:, :]) + (_dot(ev8[...], pw1[:M, :]) + pb1[...]))
    x = _relu(_dot(x, pw2[...]) + pb2[...])
    x = _relu(_dot(x, pw3[...]) + pb3[...])
    pe = _dot(x, pw4[...]) + pb4[...]
    x = _relu(_dot(se[...], sw1[...]) + sb1[...])
    x = _relu(_dot(x, sw2[...]) + sb2[...])
    x = _relu(_dot(x, sw3[...]) + sb3[...])
    sev = _dot(x, sw4[...]) + sb4[...]
    spe = (1.0 - ALPHA) * pe + ALPHA * sev
    hh = _dot(h[...], wh[...]) + bh[...]
    hhf = jnp.concatenate([hh, spe], axis=1)
    y = _dot(hhf * dout, w1g[...])
    for c in range(4):
        out[c, :, :] = y[:, c * 128:(c + 1) * 128]


def _full(shape):
    return pl.BlockSpec(shape, lambda i: (0,) * len(shape))


def _embed(hp, ev8, evp, sep, cnt, W_h, b_h, pe_params, se_params, W1g):
    pe_flat, se_flat = [], []
    for (w, b) in pe_params:
        pe_flat += [w, b.reshape(1, -1)]
    for (w, b) in se_params:
        se_flat += [w, b.reshape(1, -1)]
    in_specs = [
        _full((1, M)),
        pl.BlockSpec((RB, M), lambda i: (i, 0)),
        pl.BlockSpec((RB, 16), lambda i: (i, 0)),
        pl.BlockSpec((RB, IN_DIM), lambda i: (i, 0)),
        pl.BlockSpec((RB, 128), lambda i: (i, 0)),
        _full(W_h.shape), _full((1, HIDDEN - SPE)),
    ] + [_full(a.shape) for a in pe_flat] + [_full(a.shape) for a in se_flat] + [
        _full(W1g.shape),
    ]
    return pl.pallas_call(
        _embed_body,
        grid=(GRID,),
        in_specs=in_specs,
        out_specs=pl.BlockSpec((4, RB, 128), lambda i: (0, i, 0)),
        out_shape=jax.ShapeDtypeStruct((4, NP_, 128), jnp.float32),
    )(ev8, evp, sep, hp, cnt, W_h, b_h.reshape(1, -1), *pe_flat, *se_flat, W1g)


def _layer_body(agg, cd, cs, bprev, w, out):
    din = _rsqrt_deg(cd[...])
    dout = _rsqrt_deg(cs[...])
    x = jnp.concatenate([agg[c] for c in range(4)], axis=1)
    x = _relu(x * din + bprev[...]) * dout
    y = _dot(x, w[...])
    for c in range(y.shape[1] // 128):
        out[c, :, :] = y[:, c * 128:(c + 1) * 128]


def _layer(agg4, cnt, b_prev, W):
    co = W.shape[1] // 128
    return pl.pallas_call(
        _layer_body,
        grid=(GRID,),
        in_specs=[
            pl.BlockSpec((4, RB, 128), lambda i: (0, i, 0)),
            pl.BlockSpec((RB, 128), lambda i: (i + GRID, 0)),
            pl.BlockSpec((RB, 128), lambda i: (i, 0)),
            _full((1, HIDDEN)),
            _full(W.shape),
        ],
        out_specs=pl.BlockSpec((co, RB, 128), lambda i: (0, i, 0)),
        out_shape=jax.ShapeDtypeStruct((co, NP_, 128), jnp.float32),
    )(agg4, cnt, cnt, b_prev.reshape(1, -1), W)


def _final_body(agg, cd, b, out):
    din = _rsqrt_deg(cd[...])
    x = jnp.concatenate([agg[c] for c in range(2)], axis=1)
    out[...] = x * din + b[...]


def _final(agg2, cnt, b):
    return pl.pallas_call(
        _final_body,
        grid=(GRID,),
        in_specs=[
            pl.BlockSpec((2, RB, 128), lambda i: (0, i, 0)),
            pl.BlockSpec((RB, 128), lambda i: (i + GRID, 0)),
            _full((1, OUT_DIM)),
        ],
        out_specs=pl.BlockSpec((RB, OUT_DIM), lambda i: (i, 0)),
        out_shape=jax.ShapeDtypeStruct((N, OUT_DIM), jnp.float32),
    )(agg2, cnt, b.reshape(1, -1))


_MESH = plsc.VectorSubcoreMesh(core_axis_name="c", subcore_axis_name="s")


def _bincount_kernel(idx_hbm, ones_hbm, zeros_hbm, cnt_hbm, idx2d, ones_v, acc, sem):
    cid = lax.axis_index("c")
    sid = lax.axis_index("s")
    pltpu.sync_copy(ones_hbm, ones_v)
    pltpu.sync_copy(idx_hbm.at[pl.ds((cid * NTILE + sid) * NIT, NIT)], idx2d)
    pltpu.sync_copy(zeros_hbm, acc.at[pl.ds(sid * TR, TR)])
    plsc.subcore_barrier()

    @pl.loop(0, NIT, step=8)
    def _(j):
        for b in range(8):
            pltpu.async_copy(ones_v, acc.at[idx2d.at[j + b]], sem, add=True)
        for b in range(8):
            pltpu.make_async_copy(ones_v, acc.at[idx2d.at[0]], sem).wait()

    plsc.subcore_barrier()
    pltpu.sync_copy(acc.at[pl.ds(sid * TR, TR)],
                    cnt_hbm.at[pl.ds(cid * NP_ + sid * TR, TR)])


def _bincount(idx2):
    ones = jnp.ones((K, 128), jnp.float32)
    zeros = jnp.zeros((TR, 128), jnp.float32)
    f = pl.kernel(
        _bincount_kernel,
        out_type=jax.ShapeDtypeStruct((2 * NP_, 128), jnp.float32),
        mesh=_MESH,
        scratch_types=[
            pltpu.VMEM((NIT, K), jnp.int32),
            pltpu.VMEM((K, 128), jnp.float32),
            pltpu.VMEM_SHARED((NP_, 128), jnp.float32),
            pltpu.SemaphoreType.DMA,
        ],
    )
    return f(idx2.reshape(2 * NTILE * NIT, K), ones, zeros)


def _make_segsum(C):
    CPC = C // NSC

    def body(srcoff_hbm, dst_hbm, y_hbm, zeros_hbm, out_hbm,
             src2d, dst2d, rows0, rows1, acc, gsem, ssem0, ssem1):
        cid = lax.axis_index("c")
        sid = lax.axis_index("s")
        for k in range(CPC):
            c = cid * CPC + k
            off = c * NP_
            pltpu.sync_copy(zeros_hbm, acc.at[pl.ds(sid * TR, TR)])
            plsc.subcore_barrier()

            for half in range(2):
                hrow = half * NH
                pltpu.sync_copy(
                    srcoff_hbm.at[pl.ds((c * NTILE + sid) * NIT + hrow, NH)],
                    src2d)
                pltpu.sync_copy(dst_hbm.at[pl.ds(sid * NIT + hrow, NH)], dst2d)

                @pl.loop(0, NH, step=2)
                def _(j):
                    @pl.when(j > 0)
                    def _():
                        pltpu.make_async_copy(rows0, acc.at[dst2d.at[0]], ssem0).wait()
                    pltpu.async_copy(y_hbm.at[src2d.at[j]], rows0, gsem).wait()
                    pltpu.async_copy(rows0, acc.at[dst2d.at[j]], ssem0, add=True)

                    @pl.when(j > 1)
                    def _():
                        pltpu.make_async_copy(rows1, acc.at[dst2d.at[0]], ssem1).wait()
                    pltpu.async_copy(y_hbm.at[src2d.at[j + 1]], rows1, gsem).wait()
                    pltpu.async_copy(rows1, acc.at[dst2d.at[j + 1]], ssem1, add=True)

                pltpu.make_async_copy(rows0, acc.at[dst2d.at[0]], ssem0).wait()
                pltpu.make_async_copy(rows1, acc.at[dst2d.at[0]], ssem1).wait()

            plsc.subcore_barrier()
            pltpu.sync_copy(acc.at[pl.ds(sid * TR, TR)],
                            out_hbm.at[pl.ds(off + sid * TR, TR)])

    return pl.kernel(
        body,
        out_type=jax.ShapeDtypeStruct((C * NP_, 128), jnp.float32),
        mesh=_MESH,
        scratch_types=[
            pltpu.VMEM((NH, K), jnp.int32),
            pltpu.VMEM((NH, K), jnp.int32),
            pltpu.VMEM((K, 128), jnp.float32),
            pltpu.VMEM((K, 128), jnp.float32),
            pltpu.VMEM_SHARED((NP_, 128), jnp.float32),
            pltpu.SemaphoreType.DMA,
            pltpu.SemaphoreType.DMA,
            pltpu.SemaphoreType.DMA,
        ],
    )


def _pad_rows(x, rows):
    return jnp.concatenate(
        [x, jnp.zeros((rows - x.shape[0],) + x.shape[1:], x.dtype)], axis=0)


def kernel(h, edge_index, EigVals, EigVecs, SE, W_h, b_h, pe_params, se_params, gc_params):
    src = edge_index[0].astype(jnp.int32)
    dst = edge_index[1].astype(jnp.int32)
    padn = EP - E
    srcp = jnp.concatenate([src, jnp.full((padn,), NP_ - 1, jnp.int32)])
    dstp = jnp.concatenate([dst, jnp.full((padn,), NP_ - 1, jnp.int32)])
    idx2 = jnp.concatenate([srcp, dstp])

    cnt = _bincount(idx2)

    order = jnp.argsort(srcp)
    srcp = srcp[order]
    dstp = dstp[order]
    srcoff = jnp.concatenate([srcp + c * NP_ for c in range(4)])

    hp = _pad_rows(h, NP_)
    evp = _pad_rows(EigVecs[:, :M], NP_)
    sep = _pad_rows(SE, NP_)
    ev8 = EigVals[:M].reshape(1, M)

    zeros = jnp.zeros((TR, 128), jnp.float32)
    seg4 = _make_segsum(4)
    seg2 = _make_segsum(2)

    y = _embed(hp, ev8, evp, sep, cnt, W_h, b_h, pe_params, se_params, gc_params[0][0])
    srcoff2d = srcoff.reshape(4 * NTILE * NIT, K)
    dstp2d = dstp.reshape(NTILE * NIT, K)
    agg = seg4(srcoff2d, dstp2d, y.reshape(4 * NP_, 128), zeros)
    for i in (1, 2):
        y = _layer(agg.reshape(4, NP_, 128), cnt, gc_params[i - 1][1], gc_params[i][0])
        agg = seg4(srcoff2d, dstp2d, y.reshape(4 * NP_, 128), zeros)
    y = _layer(agg.reshape(4, NP_, 128), cnt, gc_params[2][1], gc_params[3][0])
    agg = seg2(srcoff2d, dstp2d, y.reshape(2 * NP_, 128), zeros)
    return _final(agg.reshape(2, NP_, 128), cnt, gc_params[3][1])


# speedup vs baseline: 1.2277x; 1.2277x over previous
import jax
import jax.numpy as jnp
from jax import lax
from jax.experimental import pallas as pl
from jax.experimental.pallas import tpu as pltpu
from jax.experimental.pallas import tpu_sc as plsc

N = 10000
E = 160000
IN_DIM = 256
HIDDEN = 512
OUT_DIM = 256
SPE = 64
M = 8
ALPHA = 0.5

NP_ = 10240
RB = 512
GRID = NP_ // RB
NSC = 2
NTILE = 16
TR = NP_ // NTILE
K = 128
NIT = 80
NH = 40
EP = NTILE * NIT * K


def _relu(x):
    return jnp.maximum(x, 0.0)


def _rsqrt_deg(cnt_blk):
    return lax.rsqrt(jnp.maximum(cnt_blk[:, :1], 1.0))


def _dot(a, b):
    return jnp.dot(a, b, preferred_element_type=jnp.float32)


def _embed_body(ev8, evec, se, h, cs, wh, bh,
                pw1, pb1, pw2, pb2, pw3, pb3, pw4, pb4,
                sw1, sb1, sw2, sb2, sw3, sb3, sw4, sb4,
                w1g, out):
    dout = _rsqrt_deg(cs[...])
    x = _relu(_dot(evec[...], pw1[M:, :]) + (_dot(ev8[...], pw1[:M, :]) + pb1[...]))
    x = _relu(_dot(x, pw2[...]) + pb2[...])
    x = _relu(_dot(x, pw3[...]) + pb3[...])
    pe = _dot(x, pw4[...]) + pb4[...]
    x = _relu(_dot(se[...], sw1[...]) + sb1[...])
    x = _relu(_dot(x, sw2[...]) + sb2[...])
    x = _relu(_dot(x, sw3[...]) + sb3[...])
    sev = _dot(x, sw4[...]) + sb4[...]
    spe = (1.0 - ALPHA) * pe + ALPHA * sev
    hh = _dot(h[...], wh[...]) + bh[...]
    hhf = jnp.concatenate([hh, spe], axis=1)
    y = _dot(hhf * dout, w1g[...])
    for c in range(4):
        out[c, :, :] = y[:, c * 128:(c + 1) * 128]


def _full(shape):
    return pl.BlockSpec(shape, lambda i: (0,) * len(shape))


def _embed(hp, ev8, evp, sep, cnt, W_h, b_h, pe_params, se_params, W1g):
    pe_flat, se_flat = [], []
    for (w, b) in pe_params:
        pe_flat += [w, b.reshape(1, -1)]
    for (w, b) in se_params:
        se_flat += [w, b.reshape(1, -1)]
    in_specs = [
        _full((1, M)),
        pl.BlockSpec((RB, M), lambda i: (i, 0)),
        pl.BlockSpec((RB, 16), lambda i: (i, 0)),
        pl.BlockSpec((RB, IN_DIM), lambda i: (i, 0)),
        pl.BlockSpec((RB, 128), lambda i: (i, 0)),
        _full(W_h.shape), _full((1, HIDDEN - SPE)),
    ] + [_full(a.shape) for a in pe_flat] + [_full(a.shape) for a in se_flat] + [
        _full(W1g.shape),
    ]
    return pl.pallas_call(
        _embed_body,
        grid=(GRID,),
        in_specs=in_specs,
        out_specs=pl.BlockSpec((4, RB, 128), lambda i: (0, i, 0)),
        out_shape=jax.ShapeDtypeStruct((4, NP_, 128), jnp.float32),
    )(ev8, evp, sep, hp, cnt, W_h, b_h.reshape(1, -1), *pe_flat, *se_flat, W1g)


def _layer_body(agg, cd, cs, bprev, w, out):
    din = _rsqrt_deg(cd[...])
    dout = _rsqrt_deg(cs[...])
    x = jnp.concatenate([agg[c] for c in range(4)], axis=1)
    x = _relu(x * din + bprev[...]) * dout
    y = _dot(x, w[...])
    for c in range(y.shape[1] // 128):
        out[c, :, :] = y[:, c * 128:(c + 1) * 128]


def _layer(agg4, cnt, b_prev, W):
    co = W.shape[1] // 128
    return pl.pallas_call(
        _layer_body,
        grid=(GRID,),
        in_specs=[
            pl.BlockSpec((4, RB, 128), lambda i: (0, i, 0)),
            pl.BlockSpec((RB, 128), lambda i: (i + GRID, 0)),
            pl.BlockSpec((RB, 128), lambda i: (i, 0)),
            _full((1, HIDDEN)),
            _full(W.shape),
        ],
        out_specs=pl.BlockSpec((co, RB, 128), lambda i: (0, i, 0)),
        out_shape=jax.ShapeDtypeStruct((co, NP_, 128), jnp.float32),
    )(agg4, cnt, cnt, b_prev.reshape(1, -1), W)


def _final_body(agg, cd, b, out):
    din = _rsqrt_deg(cd[...])
    x = jnp.concatenate([agg[c] for c in range(2)], axis=1)
    out[...] = x * din + b[...]


def _final(agg2, cnt, b):
    return pl.pallas_call(
        _final_body,
        grid=(GRID,),
        in_specs=[
            pl.BlockSpec((2, RB, 128), lambda i: (0, i, 0)),
            pl.BlockSpec((RB, 128), lambda i: (i + GRID, 0)),
            _full((1, OUT_DIM)),
        ],
        out_specs=pl.BlockSpec((RB, OUT_DIM), lambda i: (i, 0)),
        out_shape=jax.ShapeDtypeStruct((N, OUT_DIM), jnp.float32),
    )(agg2, cnt, b.reshape(1, -1))


_MESH = plsc.VectorSubcoreMesh(core_axis_name="c", subcore_axis_name="s")


def _bincount_kernel(idx_hbm, ones_hbm, zeros_hbm, cnt_hbm, idx2d, ones_v, acc, sem):
    cid = lax.axis_index("c")
    sid = lax.axis_index("s")
    pltpu.sync_copy(ones_hbm, ones_v)
    pltpu.sync_copy(idx_hbm.at[pl.ds((cid * NTILE + sid) * NIT, NIT)], idx2d)
    pltpu.sync_copy(zeros_hbm, acc.at[pl.ds(sid * TR, TR)])
    plsc.subcore_barrier()

    @pl.loop(0, NIT, step=8)
    def _(j):
        for b in range(8):
            pltpu.async_copy(ones_v, acc.at[idx2d.at[j + b]], sem, add=True)
        for b in range(8):
            pltpu.make_async_copy(ones_v, acc.at[idx2d.at[0]], sem).wait()

    plsc.subcore_barrier()
    pltpu.sync_copy(acc.at[pl.ds(sid * TR, TR)],
                    cnt_hbm.at[pl.ds(cid * NP_ + sid * TR, TR)])


def _bincount(idx2):
    ones = jnp.ones((K, 128), jnp.float32)
    zeros = jnp.zeros((TR, 128), jnp.float32)
    f = pl.kernel(
        _bincount_kernel,
        out_type=jax.ShapeDtypeStruct((2 * NP_, 128), jnp.float32),
        mesh=_MESH,
        scratch_types=[
            pltpu.VMEM((NIT, K), jnp.int32),
            pltpu.VMEM((K, 128), jnp.float32),
            pltpu.VMEM_SHARED((NP_, 128), jnp.float32),
            pltpu.SemaphoreType.DMA,
        ],
    )
    return f(idx2.reshape(2 * NTILE * NIT, K), ones, zeros)


def _make_segsum(C):
    CPC = C // NSC

    def body(srcoff_hbm, dst_hbm, y_hbm, zeros_hbm, out_hbm,
             src2d, dst2d, rows0, rows1, acc, gsem, ssem0, ssem1):
        cid = lax.axis_index("c")
        sid = lax.axis_index("s")
        for k in range(CPC):
            c = cid * CPC + k
            off = c * NP_
            pltpu.sync_copy(zeros_hbm, acc.at[pl.ds(sid * TR, TR)])
            plsc.subcore_barrier()

            for half in range(2):
                hrow = half * NH
                pltpu.sync_copy(
                    srcoff_hbm.at[pl.ds((c * NTILE + sid) * NIT + hrow, NH)],
                    src2d)
                pltpu.sync_copy(dst_hbm.at[pl.ds(sid * NIT + hrow, NH)], dst2d)

                @pl.loop(0, NH, step=2)
                def _(j):
                    @pl.when(j > 0)
                    def _():
                        pltpu.make_async_copy(rows0, acc.at[dst2d.at[0]], ssem0).wait()
                    pltpu.async_copy(y_hbm.at[src2d.at[j]], rows0, gsem).wait()
                    pltpu.async_copy(rows0, acc.at[dst2d.at[j]], ssem0, add=True)

                    @pl.when(j > 1)
                    def _():
                        pltpu.make_async_copy(rows1, acc.at[dst2d.at[0]], ssem1).wait()
                    pltpu.async_copy(y_hbm.at[src2d.at[j + 1]], rows1, gsem).wait()
                    pltpu.async_copy(rows1, acc.at[dst2d.at[j + 1]], ssem1, add=True)

                pltpu.make_async_copy(rows0, acc.at[dst2d.at[0]], ssem0).wait()
                pltpu.make_async_copy(rows1, acc.at[dst2d.at[0]], ssem1).wait()

            plsc.subcore_barrier()
            pltpu.sync_copy(acc.at[pl.ds(sid * TR, TR)],
                            out_hbm.at[pl.ds(off + sid * TR, TR)])

    return pl.kernel(
        body,
        out_type=jax.ShapeDtypeStruct((C * NP_, 128), jnp.float32),
        mesh=_MESH,
        scratch_types=[
            pltpu.VMEM((NH, K), jnp.int32),
            pltpu.VMEM((NH, K), jnp.int32),
            pltpu.VMEM((K, 128), jnp.float32),
            pltpu.VMEM((K, 128), jnp.float32),
            pltpu.VMEM_SHARED((NP_, 128), jnp.float32),
            pltpu.SemaphoreType.DMA,
            pltpu.SemaphoreType.DMA,
            pltpu.SemaphoreType.DMA,
        ],
    )


def _pad_rows(x, rows):
    return jnp.concatenate(
        [x, jnp.zeros((rows - x.shape[0],) + x.shape[1:], x.dtype)], axis=0)


def kernel(h, edge_index, EigVals, EigVecs, SE, W_h, b_h, pe_params, se_params, gc_params):
    src = edge_index[0].astype(jnp.int32)
    dst = edge_index[1].astype(jnp.int32)
    padn = EP - E
    srcp = jnp.concatenate([src, jnp.full((padn,), NP_ - 1, jnp.int32)])
    dstp = jnp.concatenate([dst, jnp.full((padn,), NP_ - 1, jnp.int32)])
    idx2 = jnp.concatenate([srcp, dstp])

    cnt = _bincount(idx2)

    srcoff = jnp.concatenate([srcp + c * NP_ for c in range(4)])

    hp = _pad_rows(h, NP_)
    evp = _pad_rows(EigVecs[:, :M], NP_)
    sep = _pad_rows(SE, NP_)
    ev8 = EigVals[:M].reshape(1, M)

    zeros = jnp.zeros((TR, 128), jnp.float32)
    seg4 = _make_segsum(4)
    seg2 = _make_segsum(2)

    y = _embed(hp, ev8, evp, sep, cnt, W_h, b_h, pe_params, se_params, gc_params[0][0])
    srcoff2d = srcoff.reshape(4 * NTILE * NIT, K)
    dstp2d = dstp.reshape(NTILE * NIT, K)
    agg = seg4(srcoff2d, dstp2d, y.reshape(4 * NP_, 128), zeros)
    for i in (1, 2):
        y = _layer(agg.reshape(4, NP_, 128), cnt, gc_params[i - 1][1], gc_params[i][0])
        agg = seg4(srcoff2d, dstp2d, y.reshape(4 * NP_, 128), zeros)
    y = _layer(agg.reshape(4, NP_, 128), cnt, gc_params[2][1], gc_params[3][0])
    agg = seg2(srcoff2d, dstp2d, y.reshape(2 * NP_, 128), zeros)
    return _final(agg.reshape(2, NP_, 128), cnt, gc_params[3][1])


# speedup vs baseline: 1.2725x; 1.0365x over previous
import jax
import jax.numpy as jnp
from jax import lax
from jax.experimental import pallas as pl
from jax.experimental.pallas import tpu as pltpu
from jax.experimental.pallas import tpu_sc as plsc

N = 10000
E = 160000
IN_DIM = 256
HIDDEN = 512
OUT_DIM = 256
SPE = 64
M = 8
ALPHA = 0.5

NP_ = 10240
RB = 512
GRID = NP_ // RB
NSC = 2
NTILE = 16
TR = NP_ // NTILE
K = 128
NIT = 80
NH = 40
EP = NTILE * NIT * K


def _relu(x):
    return jnp.maximum(x, 0.0)


def _rsqrt_deg(cnt_blk):
    return lax.rsqrt(jnp.maximum(cnt_blk[:, :1], 1.0))


def _dot(a, b):
    return jnp.dot(a, b, preferred_element_type=jnp.float32)


def _embed_body(ev8, evec, se, h, cs, wh, bh,
                pw1, pb1, pw2, pb2, pw3, pb3, pw4, pb4,
                sw1, sb1, sw2, sb2, sw3, sb3, sw4, sb4,
                w1g, out):
    dout = _rsqrt_deg(cs[...])
    x = _relu(_dot(evec[...], pw1[M:, :]) + (_dot(ev8[...], pw1[:M, :]) + pb1[...]))
    x = _relu(_dot(x, pw2[...]) + pb2[...])
    x = _relu(_dot(x, pw3[...]) + pb3[...])
    pe = _dot(x, pw4[...]) + pb4[...]
    x = _relu(_dot(se[...], sw1[...]) + sb1[...])
    x = _relu(_dot(x, sw2[...]) + sb2[...])
    x = _relu(_dot(x, sw3[...]) + sb3[...])
    sev = _dot(x, sw4[...]) + sb4[...]
    spe = (1.0 - ALPHA) * pe + ALPHA * sev
    hh = _dot(h[...], wh[...]) + bh[...]
    hhf = jnp.concatenate([hh, spe], axis=1)
    y = _dot(hhf * dout, w1g[...])
    for c in range(4):
        out[c, :, :] = y[:, c * 128:(c + 1) * 128]


def _full(shape):
    return pl.BlockSpec(shape, lambda i: (0,) * len(shape))


def _embed(hp, ev8, evp, sep, cnt, W_h, b_h, pe_params, se_params, W1g):
    pe_flat, se_flat = [], []
    for (w, b) in pe_params:
        pe_flat += [w, b.reshape(1, -1)]
    for (w, b) in se_params:
        se_flat += [w, b.reshape(1, -1)]
    in_specs = [
        _full((1, M)),
        pl.BlockSpec((RB, M), lambda i: (i, 0)),
        pl.BlockSpec((RB, 16), lambda i: (i, 0)),
        pl.BlockSpec((RB, IN_DIM), lambda i: (i, 0)),
        pl.BlockSpec((RB, 128), lambda i: (i, 0)),
        _full(W_h.shape), _full((1, HIDDEN - SPE)),
    ] + [_full(a.shape) for a in pe_flat] + [_full(a.shape) for a in se_flat] + [
        _full(W1g.shape),
    ]
    return pl.pallas_call(
        _embed_body,
        grid=(GRID,),
        in_specs=in_specs,
        out_specs=pl.BlockSpec((4, RB, 128), lambda i: (0, i, 0)),
        out_shape=jax.ShapeDtypeStruct((4, NP_, 128), jnp.float32),
    )(ev8, evp, sep, hp, cnt, W_h, b_h.reshape(1, -1), *pe_flat, *se_flat, W1g)


def _layer_body(agg, cd, cs, bprev, w, out):
    din = _rsqrt_deg(cd[...])
    dout = _rsqrt_deg(cs[...])
    x = jnp.concatenate([agg[c] for c in range(4)], axis=1)
    x = _relu(x * din + bprev[...]) * dout
    y = _dot(x, w[...])
    for c in range(y.shape[1] // 128):
        out[c, :, :] = y[:, c * 128:(c + 1) * 128]


def _layer(agg4, cnt, b_prev, W):
    co = W.shape[1] // 128
    return pl.pallas_call(
        _layer_body,
        grid=(GRID,),
        in_specs=[
            pl.BlockSpec((4, RB, 128), lambda i: (0, i, 0)),
            pl.BlockSpec((RB, 128), lambda i: (i + GRID, 0)),
            pl.BlockSpec((RB, 128), lambda i: (i, 0)),
            _full((1, HIDDEN)),
            _full(W.shape),
        ],
        out_specs=pl.BlockSpec((co, RB, 128), lambda i: (0, i, 0)),
        out_shape=jax.ShapeDtypeStruct((co, NP_, 128), jnp.float32),
    )(agg4, cnt, cnt, b_prev.reshape(1, -1), W)


def _final_body(agg, cd, b, out):
    din = _rsqrt_deg(cd[...])
    x = jnp.concatenate([agg[c] for c in range(2)], axis=1)
    out[...] = x * din + b[...]


def _final(agg2, cnt, b):
    return pl.pallas_call(
        _final_body,
        grid=(GRID,),
        in_specs=[
            pl.BlockSpec((2, RB, 128), lambda i: (0, i, 0)),
            pl.BlockSpec((RB, 128), lambda i: (i + GRID, 0)),
            _full((1, OUT_DIM)),
        ],
        out_specs=pl.BlockSpec((RB, OUT_DIM), lambda i: (i, 0)),
        out_shape=jax.ShapeDtypeStruct((N, OUT_DIM), jnp.float32),
    )(agg2, cnt, b.reshape(1, -1))


_MESH = plsc.VectorSubcoreMesh(core_axis_name="c", subcore_axis_name="s")


def _bincount_kernel(idx_hbm, ones_hbm, zeros_hbm, cnt_hbm, idx2d, ones_v, acc, sem):
    cid = lax.axis_index("c")
    sid = lax.axis_index("s")
    pltpu.sync_copy(ones_hbm, ones_v)
    pltpu.sync_copy(idx_hbm.at[pl.ds((cid * NTILE + sid) * NIT, NIT)], idx2d)
    pltpu.sync_copy(zeros_hbm, acc.at[pl.ds(sid * TR, TR)])
    plsc.subcore_barrier()

    @pl.loop(0, NIT, step=8)
    def _(j):
        for b in range(8):
            pltpu.async_copy(ones_v, acc.at[idx2d.at[j + b]], sem, add=True)
        for b in range(8):
            pltpu.make_async_copy(ones_v, acc.at[idx2d.at[0]], sem).wait()

    plsc.subcore_barrier()
    pltpu.sync_copy(acc.at[pl.ds(sid * TR, TR)],
                    cnt_hbm.at[pl.ds(cid * NP_ + sid * TR, TR)])


def _bincount(idx2):
    ones = jnp.ones((K, 128), jnp.float32)
    zeros = jnp.zeros((TR, 128), jnp.float32)
    f = pl.kernel(
        _bincount_kernel,
        out_type=jax.ShapeDtypeStruct((2 * NP_, 128), jnp.float32),
        mesh=_MESH,
        scratch_types=[
            pltpu.VMEM((NIT, K), jnp.int32),
            pltpu.VMEM((K, 128), jnp.float32),
            pltpu.VMEM_SHARED((NP_, 128), jnp.float32),
            pltpu.SemaphoreType.DMA,
        ],
    )
    return f(idx2.reshape(2 * NTILE * NIT, K), ones, zeros)


def _make_segsum(C):
    CPC = C // NSC

    def body(srcoff_hbm, dst_hbm, y_hbm, zeros_hbm, out_hbm,
             src2d, dst2d, rows0, rows1, acc, gsem, ssem0, ssem1):
        cid = lax.axis_index("c")
        sid = lax.axis_index("s")
        for k in range(CPC):
            c = cid * CPC + k
            off = c * NP_
            pltpu.sync_copy(zeros_hbm, acc.at[pl.ds(sid * TR, TR)])
            plsc.subcore_barrier()

            for half in range(2):
                hrow = half * NH
                pltpu.sync_copy(
                    srcoff_hbm.at[pl.ds((c * NTILE + sid) * NIT + hrow, NH)],
                    src2d)
                pltpu.sync_copy(dst_hbm.at[pl.ds(sid * NIT + hrow, NH)], dst2d)

                @pl.loop(0, NH, step=2)
                def _(j):
                    @pl.when(j > 0)
                    def _():
                        pltpu.make_async_copy(rows0, acc.at[dst2d.at[0]], ssem0).wait()
                    pltpu.async_copy(y_hbm.at[src2d.at[j]], rows0, gsem)

                    @pl.when(j > 1)
                    def _():
                        pltpu.make_async_copy(rows1, acc.at[dst2d.at[0]], ssem1).wait()
                    pltpu.async_copy(y_hbm.at[src2d.at[j + 1]], rows1, gsem)

                    pltpu.make_async_copy(y_hbm.at[src2d.at[0]], rows0, gsem).wait()
                    pltpu.async_copy(rows0, acc.at[dst2d.at[j]], ssem0, add=True)
                    pltpu.make_async_copy(y_hbm.at[src2d.at[0]], rows1, gsem).wait()
                    pltpu.async_copy(rows1, acc.at[dst2d.at[j + 1]], ssem1, add=True)

                pltpu.make_async_copy(rows0, acc.at[dst2d.at[0]], ssem0).wait()
                pltpu.make_async_copy(rows1, acc.at[dst2d.at[0]], ssem1).wait()

            plsc.subcore_barrier()
            pltpu.sync_copy(acc.at[pl.ds(sid * TR, TR)],
                            out_hbm.at[pl.ds(off + sid * TR, TR)])

    return pl.kernel(
        body,
        out_type=jax.ShapeDtypeStruct((C * NP_, 128), jnp.float32),
        mesh=_MESH,
        scratch_types=[
            pltpu.VMEM((NH, K), jnp.int32),
            pltpu.VMEM((NH, K), jnp.int32),
            pltpu.VMEM((K, 128), jnp.float32),
            pltpu.VMEM((K, 128), jnp.float32),
            pltpu.VMEM_SHARED((NP_, 128), jnp.float32),
            pltpu.SemaphoreType.DMA,
            pltpu.SemaphoreType.DMA,
            pltpu.SemaphoreType.DMA,
        ],
    )


def _pad_rows(x, rows):
    return jnp.concatenate(
        [x, jnp.zeros((rows - x.shape[0],) + x.shape[1:], x.dtype)], axis=0)


def kernel(h, edge_index, EigVals, EigVecs, SE, W_h, b_h, pe_params, se_params, gc_params):
    src = edge_index[0].astype(jnp.int32)
    dst = edge_index[1].astype(jnp.int32)
    padn = EP - E
    srcp = jnp.concatenate([src, jnp.full((padn,), NP_ - 1, jnp.int32)])
    dstp = jnp.concatenate([dst, jnp.full((padn,), NP_ - 1, jnp.int32)])
    idx2 = jnp.concatenate([srcp, dstp])

    cnt = _bincount(idx2)

    srcoff = jnp.concatenate([srcp + c * NP_ for c in range(4)])

    hp = _pad_rows(h, NP_)
    evp = _pad_rows(EigVecs[:, :M], NP_)
    sep = _pad_rows(SE, NP_)
    ev8 = EigVals[:M].reshape(1, M)

    zeros = jnp.zeros((TR, 128), jnp.float32)
    seg4 = _make_segsum(4)
    seg2 = _make_segsum(2)

    y = _embed(hp, ev8, evp, sep, cnt, W_h, b_h, pe_params, se_params, gc_params[0][0])
    srcoff2d = srcoff.reshape(4 * NTILE * NIT, K)
    dstp2d = dstp.reshape(NTILE * NIT, K)
    agg = seg4(srcoff2d, dstp2d, y.reshape(4 * NP_, 128), zeros)
    for i in (1, 2):
        y = _layer(agg.reshape(4, NP_, 128), cnt, gc_params[i - 1][1], gc_params[i][0])
        agg = seg4(srcoff2d, dstp2d, y.reshape(4 * NP_, 128), zeros)
    y = _layer(agg.reshape(4, NP_, 128), cnt, gc_params[2][1], gc_params[3][0])
    agg = seg2(srcoff2d, dstp2d, y.reshape(2 * NP_, 128), zeros)
    return _final(agg.reshape(2, NP_, 128), cnt, gc_params[3][1])
